# baseline (speedup 1.0000x reference)
import functools

import jax
import jax.numpy as jnp
from jax import lax
from jax.experimental import pallas as pl
from jax.experimental.pallas import tpu as pltpu
from jax.experimental.pallas import tpu_sc as plsc

N_NODES = 10000
N_EDGES = 320000
D = 128
N_GRAPHS = 64
EPS = 1e-7
MIN_NORM = 1e-15

NC, NS = 2, 16
NW = NC * NS
EPW = N_EDGES // NW
CH = 80
NCHUNK = EPW // CH
RPAD = 10240
ZR = RPAD // NS
ORT = N_NODES // NS
NBLK = 10
BLK = N_NODES // NBLK

_f32 = jnp.float32


GRP = 25
NGRP = NCHUNK // GRP

_sc_mesh = plsc.VectorSubcoreMesh(core_axis_name="c", subcore_axis_name="s")


def _edge_agg_body(m_hbm, ei, agg_out, src_v, dst_v, rows0, rows1,
                   rows2, acc, sem0, sem1, sem2, ssem0, ssem1, ssem2):
    c = lax.axis_index("c")
    s = lax.axis_index("s")
    wid = c * NS + s
    rows = (rows0, rows1, rows2)
    sems = (sem0, sem1, sem2)
    ssems = (ssem0, ssem1, ssem2)
    dummy = m_hbm.at[pl.ds(0, CH)]

    def _wait(b):
        pltpu.make_async_copy(dummy, rows[b], sems[b]).wait()

    def _gather(jj, b):
        pltpu.async_copy(m_hbm.at[src_v.at[jj]], rows[b], sems[b])

    def _scatter(jj, b):
        pltpu.async_copy(rows[b], acc.at[dst_v.at[jj]], ssems[b], add=True)

    def _wait_s(b):
        pltpu.make_async_copy(rows[b], acc.at[pl.ds(0, CH)], ssems[b]).wait()

    def _zrow(i, carry):
        for j in range(D // 16):
            rows0[i, pl.ds(j * 16, 16)] = jnp.zeros((16,), _f32)
        return carry
    lax.fori_loop(0, CH, _zrow, 0)
    for k in range(ZR // CH):
        pltpu.sync_copy(rows0, acc.at[pl.ds(s * ZR + k * CH, CH)])
    plsc.subcore_barrier()

    def group(g, carry):
        pltpu.sync_copy(ei.at[0].at[wid * NGRP + g], src_v)
        pltpu.sync_copy(ei.at[1].at[wid * NGRP + g], dst_v)
        _gather(0, 0)
        _gather(1, 1)
        _wait(0); _scatter(0, 0); _gather(2, 2)
        _wait(1); _scatter(1, 1); _wait_s(0); _gather(3, 0)
        _wait(2); _scatter(2, 2); _wait_s(1); _gather(4, 1)

        def triple(t, carry2):
            for k in range(3):
                j = 3 * t + k
                _wait(k)
                _scatter(j, k)
                _wait_s((k + 2) % 3)
                _gather(j + 2, (k + 2) % 3)
            return carry2
        lax.fori_loop(1, 1 + (GRP - 7) // 3, triple, 0)
        _wait(0); _scatter(GRP - 4, 0); _wait_s(2); _gather(GRP - 2, 2)
        _wait(1); _scatter(GRP - 3, 1); _wait_s(0); _gather(GRP - 1, 0)
        _wait(2); _scatter(GRP - 2, 2); _wait_s(1)
        _wait(0); _scatter(GRP - 1, 0); _wait_s(2)
        _wait_s(0)
        return carry
    lax.fori_loop(0, NGRP, group, 0)

    plsc.subcore_barrier()
    pltpu.sync_copy(acc.at[pl.ds(s * ZR, ZR)],
                    agg_out.at[c].at[pl.ds(s * ZR, ZR)])


_edge_agg = pl.kernel(
    _edge_agg_body,
    out_type=jax.ShapeDtypeStruct((NC, RPAD, D), _f32),
    mesh=_sc_mesh,
    scratch_types=[
        pltpu.VMEM((GRP, CH), jnp.int32),
        pltpu.VMEM((GRP, CH), jnp.int32),
        pltpu.VMEM((CH, D), _f32),
        pltpu.VMEM((CH, D), _f32),
        pltpu.VMEM((CH, D), _f32),
        pltpu.VMEM_SHARED((RPAD, D), _f32),
        pltpu.SemaphoreType.DMA,
        pltpu.SemaphoreType.DMA,
        pltpu.SemaphoreType.DMA,
        pltpu.SemaphoreType.DMA,
        pltpu.SemaphoreType.DMA,
        pltpu.SemaphoreType.DMA,
    ])


def _pre_body(x_ref, w_ref, o_ref):
    x = x_ref[...]
    x0 = x[:, 0:1]
    sq = jnp.maximum(jnp.sum(x * x, axis=1, keepdims=True) - x0 * x0, 0.0)
    ynorm = jnp.maximum(jnp.sqrt(sq), MIN_NORM)
    th = jnp.maximum(x0, 1.0 + EPS)
    arc = jnp.log(th + jnp.sqrt(jnp.maximum(th * th - 1.0, MIN_NORM)))
    u = x * (arc / ynorm)
    lane = lax.broadcasted_iota(jnp.int32, (BLK, D), 1)
    u = jnp.where(lane == 0, 0.0, u)
    m = lax.dot_general(u, w_ref[...], (((1,), (1,)), ((), ())),
                        precision=lax.Precision.DEFAULT)
    o_ref[...] = jnp.where(lane == 0, 1.0, m)


_pre = pl.pallas_call(
    _pre_body,
    grid=(NBLK,),
    in_specs=[pl.BlockSpec((BLK, D), lambda i: (i, 0)),
              pl.BlockSpec((D, D), lambda i: (0, 0))],
    out_specs=pl.BlockSpec((BLK, D), lambda i: (i, 0)),
    out_shape=jax.ShapeDtypeStruct((N_NODES, D), _f32),
)


def _mid_body(p0_ref, p1_ref, w_ref, o_ref):
    a = p0_ref[0] + p1_ref[0]
    cnt = jnp.maximum(a[:, 0:1], 1.0)
    lane = lax.broadcasted_iota(jnp.int32, (BLK, D), 1)
    r = jnp.where(lane == 0, 0.0, jnp.maximum(a / cnt, 0.0))
    m = lax.dot_general(r, w_ref[...], (((1,), (1,)), ((), ())),
                        precision=lax.Precision.DEFAULT)
    o_ref[...] = jnp.where(lane == 0, 1.0, m)


_mid = pl.pallas_call(
    _mid_body,
    grid=(NBLK,),
    in_specs=[pl.BlockSpec((1, BLK, D), lambda i: (0, i, 0)),
              pl.BlockSpec((1, BLK, D), lambda i: (1, i, 0)),
              pl.BlockSpec((D, D), lambda i: (0, 0))],
    out_specs=pl.BlockSpec((BLK, D), lambda i: (i, 0)),
    out_shape=jax.ShapeDtypeStruct((N_NODES, D), _f32),
)


def _pool_body(p0_ref, p1_ref, b_ref, w_ref, bias_ref, o_ref, acc, bacc):
    i = pl.program_id(0)

    @pl.when(i == 0)
    def _init():
        acc[...] = jnp.zeros((N_GRAPHS, D), _f32)
        bacc[...] = jnp.zeros((N_GRAPHS, 1), _f32)

    a = p0_ref[0] + p1_ref[0]
    cnt = jnp.maximum(a[:, 0:1], 1.0)
    lane = lax.broadcasted_iota(jnp.int32, (BLK, D), 1)
    r = jnp.where(lane == 0, 0.0, jnp.maximum(a / cnt, 0.0))
    b = b_ref[0, 0, :]
    g = lax.broadcasted_iota(jnp.int32, (N_GRAPHS, BLK), 0)
    oh = (g == b[None, :]).astype(_f32)
    acc[...] = acc[...] + lax.dot_general(oh, r, (((1,), (0,)), ((), ())),
                                          precision=lax.Precision.HIGHEST)
    bacc[...] = bacc[...] + jnp.sum(oh, axis=1, keepdims=True)

    @pl.when(i == NBLK - 1)
    def _fin():
        pooled = acc[...] / jnp.maximum(bacc[...], 1.0)
        o_ref[...] = lax.dot_general(pooled, w_ref[...], (((1,), (1,)), ((), ())),
                                     precision=lax.Precision.HIGHEST) + bias_ref[...]


_pool = pl.pallas_call(
    _pool_body,
    grid=(NBLK,),
    in_specs=[pl.BlockSpec((1, BLK, D), lambda i: (0, i, 0)),
              pl.BlockSpec((1, BLK, D), lambda i: (1, i, 0)),
              pl.BlockSpec((1, 1, BLK), lambda i: (i, 0, 0)),
              pl.BlockSpec((D, D), lambda i: (0, 0)),
              pl.BlockSpec((1, D), lambda i: (0, 0))],
    out_specs=pl.BlockSpec((N_GRAPHS, D), lambda i: (0, 0)),
    out_shape=jax.ShapeDtypeStruct((N_GRAPHS, D), _f32),
    scratch_shapes=[pltpu.VMEM((N_GRAPHS, D), _f32),
                    pltpu.VMEM((N_GRAPHS, 1), _f32)],
)


def kernel(x, edge_index, batch, W1, W2, W4, b4):
    ei = edge_index.reshape(2, NW * NGRP, GRP, CH)
    batch3 = batch.reshape(NBLK, 1, BLK)
    b42 = b4.reshape(1, D)

    m1 = _pre(x, W1)
    agg1 = _edge_agg(m1, ei)
    m2 = _mid(agg1, agg1, W2)
    agg2 = _edge_agg(m2, ei)
    return _pool(agg2, agg2, batch3, W4, b42)


# speedup vs baseline: 1.0068x; 1.0068x over previous
import functools

import jax
import jax.numpy as jnp
from jax import lax
from jax.experimental import pallas as pl
from jax.experimental.pallas import tpu as pltpu
from jax.experimental.pallas import tpu_sc as plsc

N_NODES = 10000
N_EDGES = 320000
D = 128
N_GRAPHS = 64
EPS = 1e-7
MIN_NORM = 1e-15

NC, NS = 2, 16
NW = NC * NS
EPW = N_EDGES // NW
CH = 80
NCHUNK = EPW // CH
RPAD = 10240
ZR = RPAD // NS
ORT = N_NODES // NS
NBLK = 10
BLK = N_NODES // NBLK

_f32 = jnp.float32


GRP = 25
NGRP = NCHUNK // GRP

_sc_mesh = plsc.VectorSubcoreMesh(core_axis_name="c", subcore_axis_name="s")


def _edge_agg_body(m_hbm, ei, agg_out, src_v, dst_v, rows0, rows1,
                   rows2, acc, sem0, sem1, sem2):
    c = lax.axis_index("c")
    s = lax.axis_index("s")
    wid = c * NS + s
    rows = (rows0, rows1, rows2)
    sems = (sem0, sem1, sem2)
    dummy = m_hbm.at[pl.ds(0, CH)]

    def _wait(b):
        pltpu.make_async_copy(dummy, rows[b], sems[b]).wait()

    def _gather(jj, b):
        pltpu.async_copy(m_hbm.at[src_v.at[jj]], rows[b], sems[b])

    def _scatter(jj, b):
        pltpu.sync_copy(rows[b], acc.at[dst_v.at[jj]], add=True)

    def _zrow(i, carry):
        for j in range(D // 16):
            rows0[i, pl.ds(j * 16, 16)] = jnp.zeros((16,), _f32)
        return carry
    lax.fori_loop(0, CH, _zrow, 0)
    for k in range(ZR // CH):
        pltpu.sync_copy(rows0, acc.at[pl.ds(s * ZR + k * CH, CH)])
    plsc.subcore_barrier()

    def group(g, carry):
        pltpu.sync_copy(ei.at[0].at[wid * NGRP + g], src_v)
        pltpu.sync_copy(ei.at[1].at[wid * NGRP + g], dst_v)
        _gather(0, 0)
        _gather(1, 1)

        def triple(t, carry2):
            for k in range(3):
                _wait(k)
                _gather(3 * t + k + 2, (k + 2) % 3)
                _scatter(3 * t + k, k)
            return carry2
        lax.fori_loop(0, (GRP - 4) // 3, triple, 0)
        _wait(0); _gather(GRP - 2, 2); _scatter(GRP - 4, 0)
        _wait(1); _gather(GRP - 1, 0); _scatter(GRP - 3, 1)
        _wait(2); _scatter(GRP - 2, 2)
        _wait(0); _scatter(GRP - 1, 0)
        return carry
    lax.fori_loop(0, NGRP, group, 0)

    plsc.subcore_barrier()
    pltpu.sync_copy(acc.at[pl.ds(s * ZR, ZR)],
                    agg_out.at[c].at[pl.ds(s * ZR, ZR)])


_edge_agg = pl.kernel(
    _edge_agg_body,
    out_type=jax.ShapeDtypeStruct((NC, RPAD, D), _f32),
    mesh=_sc_mesh,
    scratch_types=[
        pltpu.VMEM((GRP, CH), jnp.int32),
        pltpu.VMEM((GRP, CH), jnp.int32),
        pltpu.VMEM((CH, D), _f32),
        pltpu.VMEM((CH, D), _f32),
        pltpu.VMEM((CH, D), _f32),
        pltpu.VMEM_SHARED((RPAD, D), _f32),
        pltpu.SemaphoreType.DMA,
        pltpu.SemaphoreType.DMA,
        pltpu.SemaphoreType.DMA,
    ])


def _pre_body(x_ref, w_ref, o_ref):
    x = x_ref[...]
    x0 = x[:, 0:1]
    sq = jnp.maximum(jnp.sum(x * x, axis=1, keepdims=True) - x0 * x0, 0.0)
    ynorm = jnp.maximum(jnp.sqrt(sq), MIN_NORM)
    th = jnp.maximum(x0, 1.0 + EPS)
    arc = jnp.log(th + jnp.sqrt(jnp.maximum(th * th - 1.0, MIN_NORM)))
    u = x * (arc / ynorm)
    lane = lax.broadcasted_iota(jnp.int32, (BLK, D), 1)
    u = jnp.where(lane == 0, 0.0, u)
    m = lax.dot_general(u, w_ref[...], (((1,), (1,)), ((), ())),
                        precision=lax.Precision.DEFAULT)
    o_ref[...] = jnp.where(lane == 0, 1.0, m)


_pre = pl.pallas_call(
    _pre_body,
    grid=(NBLK,),
    in_specs=[pl.BlockSpec((BLK, D), lambda i: (i, 0)),
              pl.BlockSpec((D, D), lambda i: (0, 0))],
    out_specs=pl.BlockSpec((BLK, D), lambda i: (i, 0)),
    out_shape=jax.ShapeDtypeStruct((N_NODES, D), _f32),
)


def _mid_body(p0_ref, p1_ref, w_ref, o_ref):
    a = p0_ref[0] + p1_ref[0]
    cnt = jnp.maximum(a[:, 0:1], 1.0)
    lane = lax.broadcasted_iota(jnp.int32, (BLK, D), 1)
    r = jnp.where(lane == 0, 0.0, jnp.maximum(a / cnt, 0.0))
    m = lax.dot_general(r, w_ref[...], (((1,), (1,)), ((), ())),
                        precision=lax.Precision.DEFAULT)
    o_ref[...] = jnp.where(lane == 0, 1.0, m)


_mid = pl.pallas_call(
    _mid_body,
    grid=(NBLK,),
    in_specs=[pl.BlockSpec((1, BLK, D), lambda i: (0, i, 0)),
              pl.BlockSpec((1, BLK, D), lambda i: (1, i, 0)),
              pl.BlockSpec((D, D), lambda i: (0, 0))],
    out_specs=pl.BlockSpec((BLK, D), lambda i: (i, 0)),
    out_shape=jax.ShapeDtypeStruct((N_NODES, D), _f32),
)


def _pool_body(p0_ref, p1_ref, b_ref, w_ref, bias_ref, o_ref, acc, bacc):
    i = pl.program_id(0)

    @pl.when(i == 0)
    def _init():
        acc[...] = jnp.zeros((N_GRAPHS, D), _f32)
        bacc[...] = jnp.zeros((N_GRAPHS, 1), _f32)

    a = p0_ref[0] + p1_ref[0]
    cnt = jnp.maximum(a[:, 0:1], 1.0)
    lane = lax.broadcasted_iota(jnp.int32, (BLK, D), 1)
    r = jnp.where(lane == 0, 0.0, jnp.maximum(a / cnt, 0.0))
    b = b_ref[0, 0, :]
    g = lax.broadcasted_iota(jnp.int32, (N_GRAPHS, BLK), 0)
    oh = (g == b[None, :]).astype(_f32)
    acc[...] = acc[...] + lax.dot_general(oh, r, (((1,), (0,)), ((), ())),
                                          precision=lax.Precision.HIGHEST)
    bacc[...] = bacc[...] + jnp.sum(oh, axis=1, keepdims=True)

    @pl.when(i == NBLK - 1)
    def _fin():
        pooled = acc[...] / jnp.maximum(bacc[...], 1.0)
        o_ref[...] = lax.dot_general(pooled, w_ref[...], (((1,), (1,)), ((), ())),
                                     precision=lax.Precision.HIGHEST) + bias_ref[...]


_pool = pl.pallas_call(
    _pool_body,
    grid=(NBLK,),
    in_specs=[pl.BlockSpec((1, BLK, D), lambda i: (0, i, 0)),
              pl.BlockSpec((1, BLK, D), lambda i: (1, i, 0)),
              pl.BlockSpec((1, 1, BLK), lambda i: (i, 0, 0)),
              pl.BlockSpec((D, D), lambda i: (0, 0)),
              pl.BlockSpec((1, D), lambda i: (0, 0))],
    out_specs=pl.BlockSpec((N_GRAPHS, D), lambda i: (0, 0)),
    out_shape=jax.ShapeDtypeStruct((N_GRAPHS, D), _f32),
    scratch_shapes=[pltpu.VMEM((N_GRAPHS, D), _f32),
                    pltpu.VMEM((N_GRAPHS, 1), _f32)],
)


def kernel(x, edge_index, batch, W1, W2, W4, b4):
    ei = edge_index.reshape(2, NW * NGRP, GRP, CH)
    batch3 = batch.reshape(NBLK, 1, BLK)
    b42 = b4.reshape(1, D)

    m1 = _pre(x, W1)
    agg1 = _edge_agg(m1, ei)
    m2 = _mid(agg1, agg1, W2)
    agg2 = _edge_agg(m2, ei)
    return _pool(agg2, agg2, batch3, W4, b42)


# speedup vs baseline: 1.0415x; 1.0345x over previous
import functools

import jax
import jax.numpy as jnp
from jax import lax
from jax.experimental import pallas as pl
from jax.experimental.pallas import tpu as pltpu
from jax.experimental.pallas import tpu_sc as plsc

N_NODES = 10000
N_EDGES = 320000
D = 128
N_GRAPHS = 64
EPS = 1e-7
MIN_NORM = 1e-15

NC, NS = 2, 16
NW = NC * NS
EPW = N_EDGES // NW
CH = 80
NCHUNK = EPW // CH
RPAD = 10240
ZR = RPAD // NS
ORT = N_NODES // NS
NBLK = 5
BLK = N_NODES // NBLK

_f32 = jnp.float32


GRP = 25
NGRP = NCHUNK // GRP

_sc_mesh = plsc.VectorSubcoreMesh(core_axis_name="c", subcore_axis_name="s")


def _edge_agg_body(m_hbm, ei, agg_out, src_v, dst_v, rows0, rows1,
                   rows2, acc, sem0, sem1, sem2):
    c = lax.axis_index("c")
    s = lax.axis_index("s")
    wid = c * NS + s
    rows = (rows0, rows1, rows2)
    sems = (sem0, sem1, sem2)
    dummy = m_hbm.at[pl.ds(0, CH)]

    def _wait(b):
        pltpu.make_async_copy(dummy, rows[b], sems[b]).wait()

    def _gather(jj, b):
        pltpu.async_copy(m_hbm.at[src_v.at[jj]], rows[b], sems[b])

    def _scatter(jj, b):
        pltpu.sync_copy(rows[b], acc.at[dst_v.at[jj]], add=True)

    def _zrow(i, carry):
        for j in range(D // 16):
            rows0[i, pl.ds(j * 16, 16)] = jnp.zeros((16,), _f32)
        return carry
    lax.fori_loop(0, CH, _zrow, 0)
    for k in range(ZR // CH):
        pltpu.sync_copy(rows0, acc.at[pl.ds(s * ZR + k * CH, CH)])
    plsc.subcore_barrier()

    def group(g, carry):
        pltpu.sync_copy(ei.at[0].at[wid * NGRP + g], src_v)
        pltpu.sync_copy(ei.at[1].at[wid * NGRP + g], dst_v)
        _gather(0, 0)
        _gather(1, 1)

        def triple(t, carry2):
            for k in range(3):
                _wait(k)
                _gather(3 * t + k + 2, (k + 2) % 3)
                _scatter(3 * t + k, k)
            return carry2
        lax.fori_loop(0, (GRP - 4) // 3, triple, 0)
        _wait(0); _gather(GRP - 2, 2); _scatter(GRP - 4, 0)
        _wait(1); _gather(GRP - 1, 0); _scatter(GRP - 3, 1)
        _wait(2); _scatter(GRP - 2, 2)
        _wait(0); _scatter(GRP - 1, 0)
        return carry
    lax.fori_loop(0, NGRP, group, 0)

    plsc.subcore_barrier()
    pltpu.sync_copy(acc.at[pl.ds(s * ZR, ZR)],
                    agg_out.at[c].at[pl.ds(s * ZR, ZR)])


_edge_agg = pl.kernel(
    _edge_agg_body,
    out_type=jax.ShapeDtypeStruct((NC, RPAD, D), _f32),
    mesh=_sc_mesh,
    scratch_types=[
        pltpu.VMEM((GRP, CH), jnp.int32),
        pltpu.VMEM((GRP, CH), jnp.int32),
        pltpu.VMEM((CH, D), _f32),
        pltpu.VMEM((CH, D), _f32),
        pltpu.VMEM((CH, D), _f32),
        pltpu.VMEM_SHARED((RPAD, D), _f32),
        pltpu.SemaphoreType.DMA,
        pltpu.SemaphoreType.DMA,
        pltpu.SemaphoreType.DMA,
    ])


def _pre_body(x_ref, w_ref, o_ref):
    x = x_ref[...]
    x0 = x[:, 0:1]
    sq = jnp.maximum(jnp.sum(x * x, axis=1, keepdims=True) - x0 * x0, 0.0)
    ynorm = jnp.maximum(jnp.sqrt(sq), MIN_NORM)
    th = jnp.maximum(x0, 1.0 + EPS)
    arc = jnp.log(th + jnp.sqrt(jnp.maximum(th * th - 1.0, MIN_NORM)))
    u = x * (arc / ynorm)
    lane = lax.broadcasted_iota(jnp.int32, (BLK, D), 1)
    u = jnp.where(lane == 0, 0.0, u)
    m = lax.dot_general(u, w_ref[...], (((1,), (1,)), ((), ())),
                        precision=lax.Precision.DEFAULT)
    o_ref[...] = jnp.where(lane == 0, 1.0, m)


_pre = pl.pallas_call(
    _pre_body,
    grid=(NBLK,),
    in_specs=[pl.BlockSpec((BLK, D), lambda i: (i, 0)),
              pl.BlockSpec((D, D), lambda i: (0, 0))],
    out_specs=pl.BlockSpec((BLK, D), lambda i: (i, 0)),
    out_shape=jax.ShapeDtypeStruct((N_NODES, D), _f32),
)


def _mid_body(p0_ref, p1_ref, w_ref, o_ref):
    a = p0_ref[0] + p1_ref[0]
    cnt = jnp.maximum(a[:, 0:1], 1.0)
    lane = lax.broadcasted_iota(jnp.int32, (BLK, D), 1)
    r = jnp.where(lane == 0, 0.0, jnp.maximum(a / cnt, 0.0))
    m = lax.dot_general(r, w_ref[...], (((1,), (1,)), ((), ())),
                        precision=lax.Precision.DEFAULT)
    o_ref[...] = jnp.where(lane == 0, 1.0, m)


_mid = pl.pallas_call(
    _mid_body,
    grid=(NBLK,),
    in_specs=[pl.BlockSpec((1, BLK, D), lambda i: (0, i, 0)),
              pl.BlockSpec((1, BLK, D), lambda i: (1, i, 0)),
              pl.BlockSpec((D, D), lambda i: (0, 0))],
    out_specs=pl.BlockSpec((BLK, D), lambda i: (i, 0)),
    out_shape=jax.ShapeDtypeStruct((N_NODES, D), _f32),
)


def _pool_body(p0_ref, p1_ref, b_ref, w_ref, bias_ref, o_ref, acc, bacc):
    i = pl.program_id(0)

    @pl.when(i == 0)
    def _init():
        acc[...] = jnp.zeros((N_GRAPHS, D), _f32)
        bacc[...] = jnp.zeros((N_GRAPHS, 1), _f32)

    a = p0_ref[0] + p1_ref[0]
    cnt = jnp.maximum(a[:, 0:1], 1.0)
    lane = lax.broadcasted_iota(jnp.int32, (BLK, D), 1)
    r = jnp.where(lane == 0, 0.0, jnp.maximum(a / cnt, 0.0))
    b = b_ref[0, 0, :]
    g = lax.broadcasted_iota(jnp.int32, (N_GRAPHS, BLK), 0)
    oh = (g == b[None, :]).astype(_f32)
    acc[...] = acc[...] + lax.dot_general(oh, r, (((1,), (0,)), ((), ())),
                                          precision=lax.Precision.DEFAULT)
    bacc[...] = bacc[...] + jnp.sum(oh, axis=1, keepdims=True)

    @pl.when(i == NBLK - 1)
    def _fin():
        pooled = acc[...] / jnp.maximum(bacc[...], 1.0)
        o_ref[...] = lax.dot_general(pooled, w_ref[...], (((1,), (1,)), ((), ())),
                                     precision=lax.Precision.HIGHEST) + bias_ref[...]


_pool = pl.pallas_call(
    _pool_body,
    grid=(NBLK,),
    in_specs=[pl.BlockSpec((1, BLK, D), lambda i: (0, i, 0)),
              pl.BlockSpec((1, BLK, D), lambda i: (1, i, 0)),
              pl.BlockSpec((1, 1, BLK), lambda i: (i, 0, 0)),
              pl.BlockSpec((D, D), lambda i: (0, 0)),
              pl.BlockSpec((1, D), lambda i: (0, 0))],
    out_specs=pl.BlockSpec((N_GRAPHS, D), lambda i: (0, 0)),
    out_shape=jax.ShapeDtypeStruct((N_GRAPHS, D), _f32),
    scratch_shapes=[pltpu.VMEM((N_GRAPHS, D), _f32),
                    pltpu.VMEM((N_GRAPHS, 1), _f32)],
)


def kernel(x, edge_index, batch, W1, W2, W4, b4):
    ei = edge_index.reshape(2, NW * NGRP, GRP, CH)
    batch3 = batch.reshape(NBLK, 1, BLK)
    b42 = b4.reshape(1, D)

    m1 = _pre(x, W1)
    agg1 = _edge_agg(m1, ei)
    m2 = _mid(agg1, agg1, W2)
    agg2 = _edge_agg(m2, ei)
    return _pool(agg2, agg2, batch3, W4, b42)
